# baseline (speedup 1.0000x reference)
import jax
import jax.numpy as jnp
from jax import lax
from jax.experimental import pallas as pl
from jax.experimental.pallas import tpu as pltpu
from jax.experimental.pallas import tpu_sc as plsc

N = 16384
C = 26
K = 100
W = C * K
N_SC = 4096
N_TC = N - N_SC

NC, NS, L = 2, 16, 16
NW = NC * NS
RPW = N_SC // NW
PPW = RPW * C
R = 8
GP = R * C
NG = GP // L
CH = RPW // R
NBUF = 4

RB = 512


def _onehot_sc_body(x_hbm, cards_hbm, out_hbm, xv, cards_v, *scr):
    bufs, sems = scr[:NBUF], scr[NBUF:]
    wid = lax.axis_index("s") * NC + lax.axis_index("c")
    pbase = pl.multiple_of(wid * PPW, PPW)
    rbase = pl.multiple_of(wid * RPW, RPW)

    pltpu.sync_copy(x_hbm.at[pl.ds(pbase, PPW)], xv)
    pltpu.sync_copy(cards_hbm, cards_v)

    zeros16 = jnp.zeros((L,), jnp.float32)
    ones16 = jnp.ones((L,), jnp.float32)
    iota16 = lax.iota(jnp.int32, L)

    NB = W // 64
    def zbody(i, _):
        r = i // NB
        base = (i % NB) * 64
        for j in range(4):
            for buf in bufs:
                buf[r, pl.ds(base + j * L, L)] = zeros16
        return 0
    lax.fori_loop(0, R * NB, zbody, 0)
    def ztail(r, _):
        for off in (NB * 64, NB * 64 + L, W - L):
            for buf in bufs:
                buf[r, pl.ds(off, L)] = zeros16
        return 0
    lax.fori_loop(0, R, ztail, 0)

    def scatter(ch, buf, val, mask_valid):
        local0 = ch * GP
        for g in range(NG):
            vals = xv[pl.ds(local0 + g * L, L)]
            pic = g * L + iota16
            rows = pic // C
            cols = (pic % C) * K + vals
            if mask_valid:
                cards_l = cards_v[pl.ds(g * L, L)]
                plsc.store_scatter(buf, [rows, cols], val, mask=vals < cards_l)
            else:
                plsc.store_scatter(buf, [rows, cols], val)

    def start_out(ch, buf, sem):
        row0 = pl.multiple_of(rbase + ch * R, R)
        pltpu.async_copy(buf, out_hbm.at[pl.ds(row0, R)], sem)

    def wait_out(buf, sem):
        pltpu.make_async_copy(buf, out_hbm.at[pl.ds(rbase, R)], sem).wait()

    for b in range(NBUF):
        scatter(b, bufs[b], ones16, True)
        start_out(b, bufs[b], sems[b])

    def step(s, _):
        ch0 = NBUF + s * NBUF
        for b in range(NBUF):
            ch = ch0 + b
            wait_out(bufs[b], sems[b])
            scatter(ch - NBUF, bufs[b], zeros16, False)
            scatter(ch, bufs[b], ones16, True)
            start_out(ch, bufs[b], sems[b])
        return 0
    lax.fori_loop(0, (CH - NBUF) // NBUF, step, 0)

    for b in range(NBUF):
        wait_out(bufs[b], sems[b])


def _onehot_tc_body(x_ref, f_ref, cls_ref, valid_ref, prev_ref, out_ref):
    del prev_ref
    xe = jnp.dot(x_ref[...].astype(jnp.float32), f_ref[...],
                 preferred_element_type=jnp.float32)
    eq = (xe == cls_ref[...]).astype(jnp.float32)
    out_ref[...] = eq * valid_ref[...]


@jax.jit
def _onehot(x_sc_flat, cards_rep, x_tc, f_mat, cls_row, valid_row):
    mesh = plsc.VectorSubcoreMesh(core_axis_name="c", subcore_axis_name="s")
    sc_call = pl.kernel(
        _onehot_sc_body,
        out_type=jax.ShapeDtypeStruct((N, W), jnp.float32),
        mesh=mesh,
        compiler_params=pltpu.CompilerParams(
            needs_layout_passes=False, use_tc_tiling_on_sc=True),
        scratch_types=[
            pltpu.VMEM((PPW,), jnp.int32),
            pltpu.VMEM((GP,), jnp.int32),
        ] + [pltpu.VMEM((R, W), jnp.float32) for _ in range(NBUF)]
          + [pltpu.SemaphoreType.DMA for _ in range(NBUF)],
    )
    out_sc = sc_call(x_sc_flat, cards_rep)
    return pl.pallas_call(
        _onehot_tc_body,
        out_shape=jax.ShapeDtypeStruct((N, W), jnp.float32),
        grid=(N_TC // RB,),
        in_specs=[
            pl.BlockSpec((RB, C), lambda i: (i, 0)),
            pl.BlockSpec((C, W), lambda i: (0, 0)),
            pl.BlockSpec((1, W), lambda i: (0, 0)),
            pl.BlockSpec((1, W), lambda i: (0, 0)),
            pl.BlockSpec(memory_space=pl.ANY),
        ],
        out_specs=pl.BlockSpec((RB, W), lambda i: (N_SC // RB + i, 0)),
        input_output_aliases={4: 0},
        compiler_params=pltpu.CompilerParams(
            dimension_semantics=("arbitrary",)),
    )(x_tc, f_mat, cls_row, valid_row, out_sc)


def kernel(x, cardinalities):
    x = x.astype(jnp.int32)
    cards = jnp.asarray(cardinalities, jnp.int32)

    x_sc_flat = x[:N_SC].reshape(N_SC * C)
    cards_rep = jnp.tile(cards, R)

    col = jnp.arange(W, dtype=jnp.int32)
    field = col // K
    cls_row = (col % K).astype(jnp.float32)[None, :]
    f_mat = (field[None, :] == jnp.arange(C, dtype=jnp.int32)[:, None]
             ).astype(jnp.float32)
    valid_row = ((col % K) < cards[field]).astype(jnp.float32)[None, :]
    return _onehot(x_sc_flat, cards_rep, x[N_SC:], f_mat, cls_row, valid_row)


# speedup vs baseline: 1.0024x; 1.0024x over previous
import jax
import jax.numpy as jnp
from jax import lax
from jax.experimental import pallas as pl
from jax.experimental.pallas import tpu as pltpu
from jax.experimental.pallas import tpu_sc as plsc

N = 16384
C = 26
K = 100
W = C * K
N_SC = 8192
N_TC = N - N_SC

NC, NS, L = 2, 16, 16
NW = NC * NS
RPW = N_SC // NW
PPW = RPW * C
R = 8
GP = R * C
NG = GP // L
CH = RPW // R
NBUF = 4

RB = 512


def _onehot_sc_body(x_hbm, cards_hbm, out_hbm, xv, cards_v, *scr):
    bufs, sems = scr[:NBUF], scr[NBUF:]
    wid = lax.axis_index("s") * NC + lax.axis_index("c")
    pbase = pl.multiple_of(wid * PPW, PPW)
    rbase = pl.multiple_of(wid * RPW, RPW)

    pltpu.sync_copy(x_hbm.at[pl.ds(pbase, PPW)], xv)
    pltpu.sync_copy(cards_hbm, cards_v)

    zeros16 = jnp.zeros((L,), jnp.float32)
    ones16 = jnp.ones((L,), jnp.float32)
    iota16 = lax.iota(jnp.int32, L)

    NB = W // 64
    def zbody(i, _):
        r = i // NB
        base = (i % NB) * 64
        for j in range(4):
            for buf in bufs:
                buf[r, pl.ds(base + j * L, L)] = zeros16
        return 0
    lax.fori_loop(0, R * NB, zbody, 0)
    def ztail(r, _):
        for off in (NB * 64, NB * 64 + L, W - L):
            for buf in bufs:
                buf[r, pl.ds(off, L)] = zeros16
        return 0
    lax.fori_loop(0, R, ztail, 0)

    def scatter(ch, buf, val, mask_valid):
        local0 = ch * GP
        for g in range(NG):
            vals = xv[pl.ds(local0 + g * L, L)]
            pic = g * L + iota16
            rows = pic // C
            cols = (pic % C) * K + vals
            if mask_valid:
                cards_l = cards_v[pl.ds(g * L, L)]
                plsc.store_scatter(buf, [rows, cols], val, mask=vals < cards_l)
            else:
                plsc.store_scatter(buf, [rows, cols], val)

    def start_out(ch, buf, sem):
        row0 = pl.multiple_of(rbase + ch * R, R)
        pltpu.async_copy(buf, out_hbm.at[pl.ds(row0, R)], sem)

    def wait_out(buf, sem):
        pltpu.make_async_copy(buf, out_hbm.at[pl.ds(rbase, R)], sem).wait()

    for b in range(NBUF):
        scatter(b, bufs[b], ones16, True)
        start_out(b, bufs[b], sems[b])

    def step(s, _):
        ch0 = NBUF + s * NBUF
        for b in range(NBUF):
            ch = ch0 + b
            wait_out(bufs[b], sems[b])
            scatter(ch - NBUF, bufs[b], zeros16, False)
            scatter(ch, bufs[b], ones16, True)
            start_out(ch, bufs[b], sems[b])
        return 0
    lax.fori_loop(0, (CH - NBUF) // NBUF, step, 0)

    for b in range(NBUF):
        wait_out(bufs[b], sems[b])


def _onehot_tc_body(x_ref, f_ref, cls_ref, valid_ref, prev_ref, out_ref):
    del prev_ref
    xe = jnp.dot(x_ref[...].astype(jnp.float32), f_ref[...],
                 preferred_element_type=jnp.float32)
    eq = (xe == cls_ref[...]).astype(jnp.float32)
    out_ref[...] = eq * valid_ref[...]


@jax.jit
def _onehot(x_sc_flat, cards_rep, x_tc, f_mat, cls_row, valid_row):
    mesh = plsc.VectorSubcoreMesh(core_axis_name="c", subcore_axis_name="s")
    sc_call = pl.kernel(
        _onehot_sc_body,
        out_type=jax.ShapeDtypeStruct((N, W), jnp.float32),
        mesh=mesh,
        compiler_params=pltpu.CompilerParams(
            needs_layout_passes=False, use_tc_tiling_on_sc=True),
        scratch_types=[
            pltpu.VMEM((PPW,), jnp.int32),
            pltpu.VMEM((GP,), jnp.int32),
        ] + [pltpu.VMEM((R, W), jnp.float32) for _ in range(NBUF)]
          + [pltpu.SemaphoreType.DMA for _ in range(NBUF)],
    )
    out_sc = sc_call(x_sc_flat, cards_rep)
    return pl.pallas_call(
        _onehot_tc_body,
        out_shape=jax.ShapeDtypeStruct((N, W), jnp.float32),
        grid=(N_TC // RB,),
        in_specs=[
            pl.BlockSpec((RB, C), lambda i: (i, 0)),
            pl.BlockSpec((C, W), lambda i: (0, 0)),
            pl.BlockSpec((1, W), lambda i: (0, 0)),
            pl.BlockSpec((1, W), lambda i: (0, 0)),
            pl.BlockSpec(memory_space=pl.ANY),
        ],
        out_specs=pl.BlockSpec((RB, W), lambda i: (N_SC // RB + i, 0)),
        input_output_aliases={4: 0},
        compiler_params=pltpu.CompilerParams(
            dimension_semantics=("arbitrary",)),
    )(x_tc, f_mat, cls_row, valid_row, out_sc)


def kernel(x, cardinalities):
    x = x.astype(jnp.int32)
    cards = jnp.asarray(cardinalities, jnp.int32)

    x_sc_flat = x[:N_SC].reshape(N_SC * C)
    cards_rep = jnp.tile(cards, R)

    col = jnp.arange(W, dtype=jnp.int32)
    field = col // K
    cls_row = (col % K).astype(jnp.float32)[None, :]
    f_mat = (field[None, :] == jnp.arange(C, dtype=jnp.int32)[:, None]
             ).astype(jnp.float32)
    valid_row = ((col % K) < cards[field]).astype(jnp.float32)[None, :]
    return _onehot(x_sc_flat, cards_rep, x[N_SC:], f_mat, cls_row, valid_row)


# speedup vs baseline: 2.9930x; 2.9857x over previous
import jax
import jax.numpy as jnp
from jax import lax
from jax.experimental import pallas as pl
from jax.experimental.pallas import tpu as pltpu
from jax.experimental.pallas import tpu_sc as plsc

N = 16384
C = 26
K = 100
W = C * K
NC, NS, L = 2, 16, 16
NW = NC * NS
RPW = N // NW
FP = C // 2
CB = 2 * K
RCH = 256
NRC = RPW // RCH
CH = FP * NRC


def _onehot_body(xt_hbm, cards_hbm, out_hbm, xv, cards_v, buf0, buf1,
                 sem0, sem1):
    bufs = (buf0, buf1)
    sems = (sem0, sem1)
    wid = lax.axis_index("s") * NC + lax.axis_index("c")
    rbase = pl.multiple_of(wid * RPW, RPW)

    pltpu.sync_copy(xt_hbm.at[:, pl.ds(rbase, RPW)], xv)
    pltpu.sync_copy(cards_hbm, cards_v)

    zeros16 = jnp.zeros((L,), jnp.float32)
    ones16 = jnp.ones((L,), jnp.float32)
    iota16 = lax.iota(jnp.int32, L)

    def zbody(r, _):
        for j in range(RCH // L):
            buf0[r, pl.ds(j * L, L)] = zeros16
            buf1[r, pl.ds(j * L, L)] = zeros16
        return 0
    lax.fori_loop(0, CB, zbody, 0)

    def scatter(ch, buf, val, mask_valid):
        fp = ch // NRC
        r0 = (ch % NRC) * RCH
        for fld in range(2):
            f = fp * 2 + fld
            for g in range(RCH // L):
                vals = xv[f, pl.ds(r0 + g * L, L)]
                rows = vals + fld * K if fld else vals
                cols = g * L + iota16
                if mask_valid:
                    cards_l = cards_v[f, :]
                    plsc.store_scatter(buf, [rows, cols], val,
                                       mask=vals < cards_l)
                else:
                    plsc.store_scatter(buf, [rows, cols], val)

    def start_out(ch, buf, sem):
        fp = ch // NRC
        row0 = pl.multiple_of(fp * CB, 8)
        col0 = pl.multiple_of(rbase + (ch % NRC) * RCH, RCH)
        pltpu.async_copy(buf, out_hbm.at[pl.ds(row0, CB), pl.ds(col0, RCH)],
                         sem)

    def wait_out(buf, sem):
        pltpu.make_async_copy(
            buf, out_hbm.at[pl.ds(0, CB), pl.ds(rbase, RCH)], sem).wait()

    for b in range(2):
        scatter(b, bufs[b], ones16, True)
        start_out(b, bufs[b], sems[b])

    def step(s, _):
        ch0 = 2 + s * 2
        for b in range(2):
            ch = ch0 + b
            wait_out(bufs[b], sems[b])
            scatter(ch - 2, bufs[b], zeros16, False)
            scatter(ch, bufs[b], ones16, True)
            start_out(ch, bufs[b], sems[b])
        return 0
    lax.fori_loop(0, (CH - 2) // 2, step, 0)

    for b in range(2):
        wait_out(bufs[b], sems[b])


@jax.jit
def _onehot_sc(xt, cards_b):
    mesh = plsc.VectorSubcoreMesh(core_axis_name="c", subcore_axis_name="s")
    f = pl.kernel(
        _onehot_body,
        out_type=jax.ShapeDtypeStruct((W, N), jnp.float32),
        mesh=mesh,
        compiler_params=pltpu.CompilerParams(
            needs_layout_passes=False, use_tc_tiling_on_sc=True),
        scratch_types=[
            pltpu.VMEM((C, RPW), jnp.int32),
            pltpu.VMEM((C, L), jnp.int32),
            pltpu.VMEM((CB, RCH), jnp.float32),
            pltpu.VMEM((CB, RCH), jnp.float32),
            pltpu.SemaphoreType.DMA,
            pltpu.SemaphoreType.DMA,
        ],
    )
    return f(xt, cards_b)


def kernel(x, cardinalities):
    xt = x.astype(jnp.int32).T
    cards = jnp.asarray(cardinalities, jnp.int32)
    cards_b = jnp.tile(cards[:, None], (1, L))
    out_t = _onehot_sc(xt, cards_b)
    return out_t.T
